# Initial kernel scaffold; baseline (speedup 1.0000x reference)
import jax
import jax.numpy as jnp
from jax.experimental import pallas as pl


def kernel(x, edge_index, W1, b1, W2, b2):
    raise NotImplementedError("write your pallas kernel here")


# speedup vs baseline: 27.3011x; 27.3011x over previous
import functools

import jax
import jax.numpy as jnp
from jax import lax
from jax.experimental import pallas as pl
from jax.experimental.pallas import tpu as pltpu
from jax.experimental.pallas import tpu_sc as plsc

N = 10000
E = 320000
NC, NS = 2, 16
NT = NC * NS
B = 128
NBLK = -(-E // (NT * B))
EPAD = NT * NBLK * B
R = 10240
RT = R // NS
DUMMY = N + 100

_f32 = jnp.float32


def _sc_mesh():
    return plsc.VectorSubcoreMesh(core_axis_name="c", subcore_axis_name="s")


def _make_deg_kernel():
    @functools.partial(
        pl.kernel,
        mesh=_sc_mesh(),
        compiler_params=pltpu.CompilerParams(use_tc_tiling_on_sc=False),
        out_type=jax.ShapeDtypeStruct((NC, R), _f32),
        scratch_types=[
            pltpu.VMEM((NBLK, B), jnp.int32),
            pltpu.VMEM((B,), _f32),
            pltpu.VMEM((RT,), _f32),
            pltpu.VMEM_SHARED((R,), _f32),
        ],
    )
    def deg_kernel(dstb, zeros, out, idx_d, ones, buf, acc):
        c = lax.axis_index("c")
        s = lax.axis_index("s")
        wid = c * NS + s
        pltpu.sync_copy(dstb.at[wid], idx_d)
        for i in range(B // 16):
            ones[pl.ds(i * 16, 16)] = jnp.ones((16,), _f32)
        r0 = s * RT
        pltpu.sync_copy(zeros.at[pl.ds(r0, RT)], buf)
        pltpu.sync_copy(buf, acc.at[pl.ds(r0, RT)])
        plsc.subcore_barrier()

        def step(j, carry):
            pltpu.sync_copy(ones, acc.at[idx_d.at[j]], add=True)
            return carry

        lax.fori_loop(0, NBLK, step, 0)
        plsc.subcore_barrier()
        pltpu.sync_copy(acc.at[pl.ds(r0, RT)], buf)
        pltpu.sync_copy(buf, out.at[c, pl.ds(r0, RT)])

    return deg_kernel


def _make_agg_kernel(D):
    @functools.partial(
        pl.kernel,
        mesh=_sc_mesh(),
        compiler_params=pltpu.CompilerParams(use_tc_tiling_on_sc=False),
        out_type=jax.ShapeDtypeStruct((NC, R, D), _f32),
        scratch_types=[
            pltpu.VMEM((NBLK, B), jnp.int32),
            pltpu.VMEM((NBLK, B), jnp.int32),
            pltpu.VMEM((B, D), _f32),
            pltpu.VMEM((RT, D), _f32),
            pltpu.VMEM_SHARED((R, D), _f32),
        ],
    )
    def agg_kernel(table, srcb, dstb, zeros, out, idx_s, idx_d, rows, buf, acc):
        c = lax.axis_index("c")
        s = lax.axis_index("s")
        wid = c * NS + s
        pltpu.sync_copy(srcb.at[wid], idx_s)
        pltpu.sync_copy(dstb.at[wid], idx_d)
        r0 = s * RT
        pltpu.sync_copy(zeros.at[pl.ds(r0, RT)], buf)
        pltpu.sync_copy(buf, acc.at[pl.ds(r0, RT)])
        plsc.subcore_barrier()

        def step(j, carry):
            pltpu.sync_copy(table.at[idx_s.at[j]], rows)
            pltpu.sync_copy(rows, acc.at[idx_d.at[j]], add=True)
            return carry

        lax.fori_loop(0, NBLK, step, 0)
        plsc.subcore_barrier()
        pltpu.sync_copy(acc.at[pl.ds(r0, RT)], buf)
        pltpu.sync_copy(buf, out.at[c, pl.ds(r0, RT)])

    return agg_kernel


def _dis_tc(deg2):
    def body(dref, oref):
        d = dref[0] + dref[1] + 1.0
        oref[...] = lax.rsqrt(d)

    return pl.pallas_call(
        body, out_shape=jax.ShapeDtypeStruct((R // 128, 128), _f32)
    )(deg2)


def _mm1_tc(x, W1, dis_col):
    def body(x_ref, w_ref, dis_ref, o_ref):
        o_ref[...] = (
            jnp.dot(x_ref[...], w_ref[...], preferred_element_type=_f32)
            * dis_ref[...]
        )

    return pl.pallas_call(
        body, out_shape=jax.ShapeDtypeStruct((N, W1.shape[1]), _f32)
    )(x, W1, dis_col)


def _mm2_tc(p, xws1, dis_col, b1r, W2):
    def body(p_ref, t_ref, dis_ref, b_ref, w_ref, o_ref):
        full = p_ref[0, :N, :] + p_ref[1, :N, :] + t_ref[...]
        h = jnp.maximum(dis_ref[...] * full + b_ref[...], 0.0)
        o_ref[...] = (
            jnp.dot(h, w_ref[...], preferred_element_type=_f32) * dis_ref[...]
        )

    return pl.pallas_call(
        body, out_shape=jax.ShapeDtypeStruct((N, W2.shape[1]), _f32)
    )(p, xws1, dis_col, b1r, W2)


def _mm3_tc(q, xws2, dis_col, b2r):
    def body(q_ref, t_ref, dis_ref, b_ref, o_ref):
        full = q_ref[0, :N, :] + q_ref[1, :N, :] + t_ref[...]
        o_ref[...] = dis_ref[...] * full + b_ref[...]

    return pl.pallas_call(
        body, out_shape=jax.ShapeDtypeStruct(xws2.shape, _f32)
    )(q, xws2, dis_col, b2r)


_deg_kernel = _make_deg_kernel()
_agg16 = _make_agg_kernel(16)
_agg40 = _make_agg_kernel(40)


def kernel(x, edge_index, W1, b1, W2, b2):
    ei = edge_index.astype(jnp.int32)
    pad = EPAD - E
    src = jnp.concatenate([ei[0], jnp.zeros((pad,), jnp.int32)])
    dst = jnp.concatenate([ei[1], jnp.full((pad,), DUMMY, jnp.int32)])
    srcb = src.reshape(NT, NBLK, B)
    dstb = dst.reshape(NT, NBLK, B)

    deg = _deg_kernel(dstb, jnp.zeros((R,), _f32))
    dis = _dis_tc(deg.reshape(NC, R // 128, 128)[...])
    dis_col = dis.reshape(R)[:N, None]

    xws1 = _mm1_tc(x, W1, dis_col)
    p = _agg16(xws1, srcb, dstb, jnp.zeros((R, 16), _f32))
    xws2 = _mm2_tc(p, xws1, dis_col, b1.reshape(1, -1), W2)
    q = _agg40(xws2, srcb, dstb, jnp.zeros((R, 40), _f32))
    out = _mm3_tc(q, xws2, dis_col, b2.reshape(1, -1))
    return out


# speedup vs baseline: 39.2825x; 1.4389x over previous
import functools

import jax
import jax.numpy as jnp
from jax import lax
from jax.experimental import pallas as pl
from jax.experimental.pallas import tpu as pltpu
from jax.experimental.pallas import tpu_sc as plsc

N = 10000
E = 320000
NC, NS = 2, 16
NT = NC * NS
B = 128
NBLK = 2 * (-(-E // (NT * B * 2)))
EPAD = NT * NBLK * B
R = 10240
RT = R // NS
DUMMY = N + 100

_f32 = jnp.float32


def _sc_mesh():
    return plsc.VectorSubcoreMesh(core_axis_name="c", subcore_axis_name="s")


def _make_deg_kernel():
    @functools.partial(
        pl.kernel,
        mesh=_sc_mesh(),
        compiler_params=pltpu.CompilerParams(use_tc_tiling_on_sc=False),
        out_type=jax.ShapeDtypeStruct((NC, R), _f32),
        scratch_types=[
            pltpu.VMEM((NBLK, B), jnp.int32),
            pltpu.VMEM((B,), _f32),
            pltpu.VMEM((RT,), _f32),
            pltpu.VMEM_SHARED((R,), _f32),
        ],
    )
    def deg_kernel(dstb, zeros, out, idx_d, ones, buf, acc):
        c = lax.axis_index("c")
        s = lax.axis_index("s")
        wid = c * NS + s
        pltpu.sync_copy(dstb.at[wid], idx_d)
        for i in range(B // 16):
            ones[pl.ds(i * 16, 16)] = jnp.ones((16,), _f32)
        r0 = s * RT
        pltpu.sync_copy(zeros.at[pl.ds(r0, RT)], buf)
        pltpu.sync_copy(buf, acc.at[pl.ds(r0, RT)])
        plsc.subcore_barrier()

        def step(j, carry):
            pltpu.sync_copy(ones, acc.at[idx_d.at[j]], add=True)
            return carry

        lax.fori_loop(0, NBLK, step, 0)
        plsc.subcore_barrier()
        pltpu.sync_copy(acc.at[pl.ds(r0, RT)], buf)
        pltpu.sync_copy(buf, out.at[c, pl.ds(r0, RT)])

    return deg_kernel


def _make_agg_kernel(D):
    @functools.partial(
        pl.kernel,
        mesh=_sc_mesh(),
        compiler_params=pltpu.CompilerParams(use_tc_tiling_on_sc=False),
        out_type=jax.ShapeDtypeStruct((NC, R, D), _f32),
        scratch_types=[
            pltpu.VMEM((NBLK, B), jnp.int32),
            pltpu.VMEM((NBLK, B), jnp.int32),
            pltpu.VMEM((B, D), _f32),
            pltpu.VMEM((B, D), _f32),
            pltpu.VMEM((RT, D), _f32),
            pltpu.VMEM_SHARED((R, D), _f32),
            pltpu.SemaphoreType.DMA,
            pltpu.SemaphoreType.DMA,
        ],
    )
    def agg_kernel(
        table, srcb, dstb, zeros, out,
        idx_s, idx_d, rows0, rows1, buf, acc, sem0, sem1,
    ):
        c = lax.axis_index("c")
        s = lax.axis_index("s")
        wid = c * NS + s
        pltpu.sync_copy(srcb.at[wid], idx_s)
        pltpu.sync_copy(dstb.at[wid], idx_d)
        r0 = s * RT
        pltpu.sync_copy(zeros.at[pl.ds(r0, RT)], buf)
        pltpu.sync_copy(buf, acc.at[pl.ds(r0, RT)])
        plsc.subcore_barrier()
        pltpu.async_copy(table.at[idx_s.at[0]], rows0, sem0)
        pltpu.async_copy(table.at[idx_s.at[1]], rows1, sem1)

        def step(t, carry):
            for off, rows, sem in ((0, rows0, sem0), (1, rows1, sem1)):
                j = 2 * t + off
                pltpu.make_async_copy(table.at[idx_s.at[j]], rows, sem).wait()
                pltpu.sync_copy(rows, acc.at[idx_d.at[j]], add=True)
                jn = lax.rem(j + 2, NBLK)
                pltpu.async_copy(table.at[idx_s.at[jn]], rows, sem)
            return carry

        lax.fori_loop(0, NBLK // 2, step, 0)
        pltpu.make_async_copy(table.at[idx_s.at[0]], rows0, sem0).wait()
        pltpu.make_async_copy(table.at[idx_s.at[1]], rows1, sem1).wait()
        plsc.subcore_barrier()
        pltpu.sync_copy(acc.at[pl.ds(r0, RT)], buf)
        pltpu.sync_copy(buf, out.at[c, pl.ds(r0, RT)])

    return agg_kernel


def _dis_tc(deg2):
    def body(dref, oref):
        d = dref[0] + dref[1] + 1.0
        oref[...] = lax.rsqrt(d)

    return pl.pallas_call(
        body, out_shape=jax.ShapeDtypeStruct((R // 128, 128), _f32)
    )(deg2)


def _mm1_tc(x, W1, dis_col):
    def body(x_ref, w_ref, dis_ref, o_ref):
        o_ref[...] = (
            jnp.dot(x_ref[...], w_ref[...], preferred_element_type=_f32)
            * dis_ref[...]
        )

    return pl.pallas_call(
        body, out_shape=jax.ShapeDtypeStruct((N, W1.shape[1]), _f32)
    )(x, W1, dis_col)


def _mm2_tc(p, xws1, dis_col, b1r):
    def body(p_ref, t_ref, dis_ref, b_ref, o_ref):
        full = p_ref[0, :N, :] + p_ref[1, :N, :] + t_ref[...]
        h = jnp.maximum(dis_ref[...] * full + b_ref[...], 0.0)
        o_ref[...] = h * dis_ref[...]

    return pl.pallas_call(
        body, out_shape=jax.ShapeDtypeStruct(xws1.shape, _f32)
    )(p, xws1, dis_col, b1r)


def _mm3_tc(q, t2, dis_col, W2, b2r):
    def body(q_ref, t_ref, dis_ref, w_ref, b_ref, o_ref):
        full = q_ref[0, :N, :] + q_ref[1, :N, :] + t_ref[...]
        o_ref[...] = (
            jnp.dot(full, w_ref[...], preferred_element_type=_f32)
            * dis_ref[...]
            + b_ref[...]
        )

    return pl.pallas_call(
        body, out_shape=jax.ShapeDtypeStruct((N, W2.shape[1]), _f32)
    )(q, t2, dis_col, W2, b2r)


_deg_kernel = _make_deg_kernel()
_agg16 = _make_agg_kernel(16)


def kernel(x, edge_index, W1, b1, W2, b2):
    ei = edge_index.astype(jnp.int32)
    pad = EPAD - E
    src = jnp.concatenate([ei[0], jnp.zeros((pad,), jnp.int32)])
    dst = jnp.concatenate([ei[1], jnp.full((pad,), DUMMY, jnp.int32)])
    srcb = src.reshape(NT, NBLK, B)
    dstb = dst.reshape(NT, NBLK, B)

    deg = _deg_kernel(dstb, jnp.zeros((R,), _f32))
    dis = _dis_tc(deg.reshape(NC, R // 128, 128)[...])
    dis_col = dis.reshape(R)[:N, None]

    z16 = jnp.zeros((R, 16), _f32)
    xws1 = _mm1_tc(x, W1, dis_col)
    p = _agg16(xws1, srcb, dstb, z16)
    t2 = _mm2_tc(p, xws1, dis_col, b1.reshape(1, -1))
    q = _agg16(t2, srcb, dstb, z16)
    out = _mm3_tc(q, t2, dis_col, W2, b2.reshape(1, -1))
    return out


# speedup vs baseline: 50.5892x; 1.2878x over previous
import functools

import jax
import jax.numpy as jnp
from jax import lax
from jax.experimental import pallas as pl
from jax.experimental.pallas import tpu as pltpu
from jax.experimental.pallas import tpu_sc as plsc

N = 10000
E = 320000
NC, NS = 2, 16
NT = NC * NS
B = 128
NBT = E // B
BASE = NBT // NT
XTRA = NBT % NT
NG = BASE // 2
R = 10240
RT = R // NS

_f32 = jnp.float32


def _sc_mesh():
    return plsc.VectorSubcoreMesh(core_axis_name="c", subcore_axis_name="s")


def _make_deg_kernel():
    @functools.partial(
        pl.kernel,
        mesh=_sc_mesh(),
        compiler_params=pltpu.CompilerParams(use_tc_tiling_on_sc=False),
        out_type=jax.ShapeDtypeStruct((NC, R), _f32),
        scratch_types=[
            pltpu.VMEM((BASE, B), jnp.int32),
            pltpu.VMEM((B,), jnp.int32),
            pltpu.VMEM((B,), _f32),
            pltpu.VMEM((RT,), _f32),
            pltpu.VMEM_SHARED((R,), _f32),
            pltpu.SemaphoreType.DMA,
        ],
    )
    def deg_kernel(eb, zeros, out, idx_d, xd, ones, buf, acc, sem):
        c = lax.axis_index("c")
        s = lax.axis_index("s")
        wid = c * NS + s
        pltpu.sync_copy(eb.at[1, pl.ds(wid * BASE, BASE)], idx_d)
        for i in range(B // 16):
            ones[pl.ds(i * 16, 16)] = jnp.ones((16,), _f32)
        r0 = s * RT
        pltpu.sync_copy(zeros, buf)
        pltpu.sync_copy(buf, acc.at[pl.ds(r0, RT)])
        plsc.subcore_barrier()

        def issue(j, carry):
            pltpu.async_copy(ones, acc.at[idx_d.at[j]], sem, add=True)
            return carry

        lax.fori_loop(0, BASE, issue, 0)

        @pl.when(wid < XTRA)
        def _():
            pltpu.sync_copy(eb.at[1, NT * BASE + wid], xd)
            pltpu.sync_copy(ones, acc.at[xd], add=True)

        def drain(j, carry):
            pltpu.make_async_copy(ones, acc.at[idx_d.at[0]], sem).wait()
            return carry

        lax.fori_loop(0, BASE, drain, 0)
        plsc.subcore_barrier()
        pltpu.sync_copy(acc.at[pl.ds(r0, RT)], buf)
        pltpu.sync_copy(buf, out.at[c, pl.ds(r0, RT)])

    return deg_kernel


def _make_agg_kernel(D):
    @functools.partial(
        pl.kernel,
        mesh=_sc_mesh(),
        compiler_params=pltpu.CompilerParams(use_tc_tiling_on_sc=False),
        out_type=jax.ShapeDtypeStruct((NC, R, D), _f32),
        scratch_types=[
            pltpu.VMEM((BASE, B), jnp.int32),
            pltpu.VMEM((BASE, B), jnp.int32),
            pltpu.VMEM((B,), jnp.int32),
            pltpu.VMEM((B,), jnp.int32),
            pltpu.VMEM((B, D), _f32),
            pltpu.VMEM((B, D), _f32),
            pltpu.VMEM((RT, D), _f32),
            pltpu.VMEM_SHARED((R, D), _f32),
            pltpu.SemaphoreType.DMA,
            pltpu.SemaphoreType.DMA,
            pltpu.SemaphoreType.DMA,
            pltpu.SemaphoreType.DMA,
        ],
    )
    def agg_kernel(
        table, eb, zeros, out,
        idx_s, idx_d, xs, xd, rows0, rows1, buf, acc, g0, g1, s0, s1,
    ):
        c = lax.axis_index("c")
        s = lax.axis_index("s")
        wid = c * NS + s
        b0 = wid * BASE
        pltpu.sync_copy(eb.at[0, pl.ds(b0, BASE)], idx_s)
        pltpu.sync_copy(eb.at[1, pl.ds(b0, BASE)], idx_d)
        r0 = s * RT
        pltpu.sync_copy(zeros, buf)
        pltpu.sync_copy(buf, acc.at[pl.ds(r0, RT)])
        plsc.subcore_barrier()
        pltpu.async_copy(table.at[idx_s.at[0]], rows0, g0)
        pltpu.async_copy(table.at[idx_s.at[1]], rows1, g1)

        def grp(t, carry):
            j0 = 2 * t
            for off, rows, gs, ss in ((0, rows0, g0, s0), (1, rows1, g1, s1)):
                pltpu.make_async_copy(table.at[idx_s.at[0]], rows, gs).wait()
                pltpu.async_copy(rows, acc.at[idx_d.at[j0 + off]], ss, add=True)
            for off, rows, gs, ss in ((2, rows0, g0, s0), (3, rows1, g1, s1)):
                jn = lax.rem(j0 + off, BASE)
                pltpu.make_async_copy(rows, acc.at[idx_d.at[0]], ss).wait()
                pltpu.async_copy(table.at[idx_s.at[jn]], rows, gs)
            return carry

        lax.fori_loop(0, NG, grp, 0)
        pltpu.make_async_copy(table.at[idx_s.at[0]], rows0, g0).wait()
        pltpu.make_async_copy(table.at[idx_s.at[1]], rows1, g1).wait()

        @pl.when(wid < XTRA)
        def _():
            pltpu.sync_copy(eb.at[0, NT * BASE + wid], xs)
            pltpu.sync_copy(eb.at[1, NT * BASE + wid], xd)
            pltpu.sync_copy(table.at[xs], rows0)
            pltpu.sync_copy(rows0, acc.at[xd], add=True)

        plsc.subcore_barrier()
        pltpu.sync_copy(acc.at[pl.ds(r0, RT)], buf)
        pltpu.sync_copy(buf, out.at[c, pl.ds(r0, RT)])

    return agg_kernel


def _dis_tc(deg2):
    def body(dref, oref):
        d = dref[0] + dref[1] + 1.0
        oref[...] = lax.rsqrt(d)

    return pl.pallas_call(
        body, out_shape=jax.ShapeDtypeStruct((R // 128, 128), _f32)
    )(deg2)


def _mm1_tc(x, W1, dis_col):
    def body(x_ref, w_ref, dis_ref, o_ref):
        o_ref[...] = (
            jnp.dot(x_ref[...], w_ref[...], preferred_element_type=_f32)
            * dis_ref[...]
        )

    return pl.pallas_call(
        body, out_shape=jax.ShapeDtypeStruct((N, W1.shape[1]), _f32)
    )(x, W1, dis_col)


def _mm2_tc(p, xws1, dis_col, b1r):
    def body(p_ref, t_ref, dis_ref, b_ref, o_ref):
        full = p_ref[0, :N, :] + p_ref[1, :N, :] + t_ref[...]
        h = jnp.maximum(dis_ref[...] * full + b_ref[...], 0.0)
        o_ref[...] = h * dis_ref[...]

    return pl.pallas_call(
        body, out_shape=jax.ShapeDtypeStruct(xws1.shape, _f32)
    )(p, xws1, dis_col, b1r)


def _mm3_tc(q, t2, dis_col, W2, b2r):
    def body(q_ref, t_ref, dis_ref, w_ref, b_ref, o_ref):
        full = q_ref[0, :N, :] + q_ref[1, :N, :] + t_ref[...]
        o_ref[...] = (
            jnp.dot(full, w_ref[...], preferred_element_type=_f32)
            * dis_ref[...]
            + b_ref[...]
        )

    return pl.pallas_call(
        body, out_shape=jax.ShapeDtypeStruct((N, W2.shape[1]), _f32)
    )(q, t2, dis_col, W2, b2r)


_deg_kernel = _make_deg_kernel()
_agg16 = _make_agg_kernel(16)


def kernel(x, edge_index, W1, b1, W2, b2):
    eb = edge_index.astype(jnp.int32).reshape(2, NBT, B)
    zr = jnp.zeros((RT,), _f32)
    z16 = jnp.zeros((RT, 16), _f32)

    deg = _deg_kernel(eb, zr)
    dis = _dis_tc(deg.reshape(NC, R // 128, 128))
    dis_col = dis.reshape(R)[:N, None]

    xws1 = _mm1_tc(x, W1, dis_col)
    p = _agg16(xws1, eb, z16)
    t2 = _mm2_tc(p, xws1, dis_col, b1.reshape(1, -1))
    q = _agg16(t2, eb, z16)
    out = _mm3_tc(q, t2, dis_col, W2, b2.reshape(1, -1))
    return out


# speedup vs baseline: 50.8553x; 1.0053x over previous
import functools

import jax
import jax.numpy as jnp
from jax import lax
from jax.experimental import pallas as pl
from jax.experimental.pallas import tpu as pltpu
from jax.experimental.pallas import tpu_sc as plsc

N = 10000
E = 320000
NC, NS = 2, 16
NT = NC * NS
B = 128
NBT = E // B
BASE = NBT // NT
XTRA = NBT % NT
NG = BASE // 2
R = 10240
RT = R // NS

_f32 = jnp.float32


def _sc_mesh():
    return plsc.VectorSubcoreMesh(core_axis_name="c", subcore_axis_name="s")


def _make_deg_kernel():
    @functools.partial(
        pl.kernel,
        mesh=_sc_mesh(),
        compiler_params=pltpu.CompilerParams(use_tc_tiling_on_sc=False),
        out_type=jax.ShapeDtypeStruct((NC, R), _f32),
        scratch_types=[
            pltpu.VMEM((BASE, B), jnp.int32),
            pltpu.VMEM((B,), jnp.int32),
            pltpu.VMEM((B,), _f32),
            pltpu.VMEM((RT,), _f32),
            pltpu.VMEM_SHARED((R,), _f32),
            pltpu.SemaphoreType.DMA,
        ],
    )
    def deg_kernel(eb, zeros, out, idx_d, xd, ones, buf, acc, sem):
        c = lax.axis_index("c")
        s = lax.axis_index("s")
        wid = c * NS + s
        pltpu.sync_copy(eb.at[1, pl.ds(wid * BASE, BASE)], idx_d)
        for i in range(B // 16):
            ones[pl.ds(i * 16, 16)] = jnp.ones((16,), _f32)
        r0 = s * RT
        pltpu.sync_copy(zeros, buf)
        pltpu.sync_copy(buf, acc.at[pl.ds(r0, RT)])
        plsc.subcore_barrier()

        def issue(j, carry):
            pltpu.async_copy(ones, acc.at[idx_d.at[j]], sem, add=True)
            return carry

        lax.fori_loop(0, BASE, issue, 0)

        @pl.when(wid < XTRA)
        def _():
            pltpu.sync_copy(eb.at[1, NT * BASE + wid], xd)
            pltpu.sync_copy(ones, acc.at[xd], add=True)

        def drain(j, carry):
            pltpu.make_async_copy(ones, acc.at[idx_d.at[0]], sem).wait()
            return carry

        lax.fori_loop(0, BASE, drain, 0)
        plsc.subcore_barrier()
        pltpu.sync_copy(acc.at[pl.ds(r0, RT)], buf)
        pltpu.sync_copy(buf, out.at[c, pl.ds(r0, RT)])

    return deg_kernel


def _make_agg_kernel(D):
    @functools.partial(
        pl.kernel,
        mesh=_sc_mesh(),
        compiler_params=pltpu.CompilerParams(use_tc_tiling_on_sc=False),
        out_type=jax.ShapeDtypeStruct((NC, R, D), _f32),
        scratch_types=[
            pltpu.VMEM((BASE, B), jnp.int32),
            pltpu.VMEM((BASE, B), jnp.int32),
            pltpu.VMEM((B,), jnp.int32),
            pltpu.VMEM((B,), jnp.int32),
            pltpu.VMEM((B, D), _f32),
            pltpu.VMEM((B, D), _f32),
            pltpu.VMEM((RT, D), _f32),
            pltpu.VMEM_SHARED((R, D), _f32),
            pltpu.SemaphoreType.DMA,
            pltpu.SemaphoreType.DMA,
            pltpu.SemaphoreType.DMA,
            pltpu.SemaphoreType.DMA,
        ],
    )
    def agg_kernel(
        table, eb, zeros, out,
        idx_s, idx_d, xs, xd, rows0, rows1, buf, acc, g0, g1, s0, s1,
    ):
        c = lax.axis_index("c")
        s = lax.axis_index("s")
        wid = c * NS + s
        b0 = wid * BASE
        pltpu.sync_copy(eb.at[0, pl.ds(b0, BASE)], idx_s)
        pltpu.sync_copy(eb.at[1, pl.ds(b0, BASE)], idx_d)
        r0 = s * RT
        pltpu.sync_copy(zeros, buf)
        pltpu.sync_copy(buf, acc.at[pl.ds(r0, RT)])
        plsc.subcore_barrier()
        pltpu.async_copy(table.at[idx_s.at[0]], rows0, g0)
        pltpu.async_copy(table.at[idx_s.at[1]], rows1, g1)

        def grp(t, carry):
            j0 = 2 * t
            for off, rows, gs, ss in ((0, rows0, g0, s0), (1, rows1, g1, s1)):
                pltpu.make_async_copy(table.at[idx_s.at[0]], rows, gs).wait()
                pltpu.async_copy(rows, acc.at[idx_d.at[j0 + off]], ss, add=True)
            for off, rows, gs, ss in ((2, rows0, g0, s0), (3, rows1, g1, s1)):
                jn = lax.rem(j0 + off, BASE)
                pltpu.make_async_copy(rows, acc.at[idx_d.at[0]], ss).wait()
                pltpu.async_copy(table.at[idx_s.at[jn]], rows, gs)
            return carry

        lax.fori_loop(0, NG, grp, 0)
        pltpu.make_async_copy(table.at[idx_s.at[0]], rows0, g0).wait()
        pltpu.make_async_copy(table.at[idx_s.at[1]], rows1, g1).wait()

        @pl.when(wid < XTRA)
        def _():
            pltpu.sync_copy(eb.at[0, NT * BASE + wid], xs)
            pltpu.sync_copy(eb.at[1, NT * BASE + wid], xd)
            pltpu.sync_copy(table.at[xs], rows0)
            pltpu.sync_copy(rows0, acc.at[xd], add=True)

        plsc.subcore_barrier()
        pltpu.sync_copy(acc.at[pl.ds(r0, RT)], buf)
        pltpu.sync_copy(buf, out.at[c, pl.ds(r0, RT)])

    return agg_kernel


_BN = 2000


def _dis_tc(deg):
    def body(dref, oref):
        d = dref[0] + dref[1] + 1.0
        oref[...] = jnp.broadcast_to(lax.rsqrt(d).reshape(R, 1)[:N], (N, 16))

    return pl.pallas_call(
        body, out_shape=jax.ShapeDtypeStruct((N, 16), _f32)
    )(deg)


def _mm1_tc(x, W1, dis16):
    def body(x_ref, w_ref, dis_ref, o_ref):
        o_ref[...] = (
            jnp.dot(x_ref[...], w_ref[...], preferred_element_type=_f32)
            * dis_ref[...]
        )

    return pl.pallas_call(
        body,
        grid=(N // _BN,),
        in_specs=[
            pl.BlockSpec((_BN, 128), lambda i: (i, 0)),
            pl.BlockSpec((128, 16), lambda i: (0, 0)),
            pl.BlockSpec((_BN, 16), lambda i: (i, 0)),
        ],
        out_specs=pl.BlockSpec((_BN, 16), lambda i: (i, 0)),
        out_shape=jax.ShapeDtypeStruct((N, W1.shape[1]), _f32),
    )(x, W1, dis16)


def _mm2_tc(p, xws1, dis16, b1r):
    def body(p_ref, t_ref, dis_ref, b_ref, o_ref):
        full = p_ref[0] + p_ref[1] + t_ref[...]
        h = jnp.maximum(dis_ref[...] * full + b_ref[...], 0.0)
        o_ref[...] = h * dis_ref[...]

    return pl.pallas_call(
        body,
        grid=(N // _BN,),
        in_specs=[
            pl.BlockSpec((2, _BN, 16), lambda i: (0, i, 0)),
            pl.BlockSpec((_BN, 16), lambda i: (i, 0)),
            pl.BlockSpec((_BN, 16), lambda i: (i, 0)),
            pl.BlockSpec((1, 16), lambda i: (0, 0)),
        ],
        out_specs=pl.BlockSpec((_BN, 16), lambda i: (i, 0)),
        out_shape=jax.ShapeDtypeStruct(xws1.shape, _f32),
    )(p, xws1, dis16, b1r)


def _mm3_tc(q, t2, dis16, W2, b2r):
    def body(q_ref, t_ref, dis_ref, w_ref, b_ref, o_ref):
        full = q_ref[0] + q_ref[1] + t_ref[...]
        o_ref[...] = (
            jnp.dot(full, w_ref[...], preferred_element_type=_f32)
            * dis_ref[:, :1]
            + b_ref[...]
        )

    return pl.pallas_call(
        body,
        grid=(N // _BN,),
        in_specs=[
            pl.BlockSpec((2, _BN, 16), lambda i: (0, i, 0)),
            pl.BlockSpec((_BN, 16), lambda i: (i, 0)),
            pl.BlockSpec((_BN, 16), lambda i: (i, 0)),
            pl.BlockSpec((16, 40), lambda i: (0, 0)),
            pl.BlockSpec((1, 40), lambda i: (0, 0)),
        ],
        out_specs=pl.BlockSpec((_BN, 40), lambda i: (i, 0)),
        out_shape=jax.ShapeDtypeStruct((N, W2.shape[1]), _f32),
    )(q, t2, dis16, W2, b2r)


_deg_kernel = _make_deg_kernel()
_agg16 = _make_agg_kernel(16)


def kernel(x, edge_index, W1, b1, W2, b2):
    eb = edge_index.astype(jnp.int32).reshape(2, NBT, B)
    zr = jnp.zeros((RT,), _f32)
    z16 = jnp.zeros((RT, 16), _f32)

    deg = _deg_kernel(eb, zr)
    dis16 = _dis_tc(deg)

    xws1 = _mm1_tc(x, W1, dis16)
    p = _agg16(xws1, eb, z16)
    t2 = _mm2_tc(p, xws1, dis16, b1.reshape(1, -1))
    q = _agg16(t2, eb, z16)
    out = _mm3_tc(q, t2, dis16, W2, b2.reshape(1, -1))
    return out


# speedup vs baseline: 56.4717x; 1.1104x over previous
import functools

import jax
import jax.numpy as jnp
from jax import lax
from jax.experimental import pallas as pl
from jax.experimental.pallas import tpu as pltpu
from jax.experimental.pallas import tpu_sc as plsc

N = 10000
E = 320000
NC, NS = 2, 16
NT = NC * NS
B = 128
NBT = E // B
BASE = NBT // NT
XTRA = NBT % NT
NG = BASE // 2
R = 10240
RT = R // NS

_f32 = jnp.float32


def _sc_mesh():
    return plsc.VectorSubcoreMesh(core_axis_name="c", subcore_axis_name="s")


def _make_deg_kernel():
    @functools.partial(
        pl.kernel,
        mesh=_sc_mesh(),
        compiler_params=pltpu.CompilerParams(use_tc_tiling_on_sc=False),
        out_type=jax.ShapeDtypeStruct((NC, R), _f32),
        scratch_types=[
            pltpu.VMEM((BASE, B), jnp.int32),
            pltpu.VMEM((B,), jnp.int32),
            pltpu.VMEM((B,), _f32),
            pltpu.VMEM((RT,), _f32),
            pltpu.VMEM_SHARED((R,), _f32),
            pltpu.SemaphoreType.DMA,
        ],
    )
    def deg_kernel(eb, zeros, out, idx_d, xd, ones, buf, acc, sem):
        c = lax.axis_index("c")
        s = lax.axis_index("s")
        wid = c * NS + s
        pltpu.sync_copy(eb.at[1, pl.ds(wid * BASE, BASE)], idx_d)
        for i in range(B // 16):
            ones[pl.ds(i * 16, 16)] = jnp.ones((16,), _f32)
        r0 = s * RT
        pltpu.sync_copy(zeros, buf)
        pltpu.sync_copy(buf, acc.at[pl.ds(r0, RT)])
        plsc.subcore_barrier()

        def issue(j, carry):
            pltpu.async_copy(ones, acc.at[idx_d.at[j]], sem, add=True)
            return carry

        lax.fori_loop(0, BASE, issue, 0)

        @pl.when(wid < XTRA)
        def _():
            pltpu.sync_copy(eb.at[1, NT * BASE + wid], xd)
            pltpu.sync_copy(ones, acc.at[xd], add=True)

        def drain(j, carry):
            pltpu.make_async_copy(ones, acc.at[idx_d.at[0]], sem).wait()
            return carry

        lax.fori_loop(0, BASE, drain, 0)
        plsc.subcore_barrier()
        pltpu.sync_copy(acc.at[pl.ds(r0, RT)], buf)
        pltpu.sync_copy(buf, out.at[c, pl.ds(r0, RT)])

    return deg_kernel


def _make_agg_kernel(D):
    @functools.partial(
        pl.kernel,
        mesh=_sc_mesh(),
        compiler_params=pltpu.CompilerParams(use_tc_tiling_on_sc=False),
        out_type=jax.ShapeDtypeStruct((NC, R, D), _f32),
        scratch_types=[
            pltpu.VMEM((BASE, B), jnp.int32),
            pltpu.VMEM((BASE, B), jnp.int32),
            pltpu.VMEM((B,), jnp.int32),
            pltpu.VMEM((B,), jnp.int32),
            pltpu.VMEM((B, D), _f32),
            pltpu.VMEM((B, D), _f32),
            pltpu.VMEM((RT, D), _f32),
            pltpu.VMEM_SHARED((R, D), _f32),
            pltpu.SemaphoreType.DMA,
            pltpu.SemaphoreType.DMA,
            pltpu.SemaphoreType.DMA,
            pltpu.SemaphoreType.DMA,
        ],
    )
    def agg_kernel(
        table, eb, zeros, out,
        idx_s, idx_d, xs, xd, rows0, rows1, buf, acc, g0, g1, s0, s1,
    ):
        c = lax.axis_index("c")
        s = lax.axis_index("s")
        wid = c * NS + s
        b0 = wid * BASE
        pltpu.sync_copy(eb.at[0, pl.ds(b0, BASE)], idx_s)
        pltpu.sync_copy(eb.at[1, pl.ds(b0, BASE)], idx_d)
        r0 = s * RT
        pltpu.sync_copy(zeros, buf)
        pltpu.sync_copy(buf, acc.at[pl.ds(r0, RT)])
        plsc.subcore_barrier()
        pltpu.async_copy(table.at[idx_s.at[0]], rows0, g0)
        pltpu.async_copy(table.at[idx_s.at[1]], rows1, g1)

        def grp(t, carry):
            j0 = 2 * t
            for off, rows, gs, ss in ((0, rows0, g0, s0), (1, rows1, g1, s1)):
                pltpu.make_async_copy(table.at[idx_s.at[0]], rows, gs).wait()
                pltpu.async_copy(rows, acc.at[idx_d.at[j0 + off]], ss, add=True)
            for off, rows, gs, ss in ((2, rows0, g0, s0), (3, rows1, g1, s1)):
                jn = lax.rem(j0 + off, BASE)
                pltpu.make_async_copy(rows, acc.at[idx_d.at[0]], ss).wait()
                pltpu.async_copy(table.at[idx_s.at[jn]], rows, gs)
            return carry

        lax.fori_loop(0, NG, grp, 0)
        pltpu.make_async_copy(table.at[idx_s.at[0]], rows0, g0).wait()
        pltpu.make_async_copy(table.at[idx_s.at[1]], rows1, g1).wait()

        @pl.when(wid < XTRA)
        def _():
            pltpu.sync_copy(eb.at[0, NT * BASE + wid], xs)
            pltpu.sync_copy(eb.at[1, NT * BASE + wid], xd)
            pltpu.sync_copy(table.at[xs], rows0)
            pltpu.sync_copy(rows0, acc.at[xd], add=True)

        plsc.subcore_barrier()
        pltpu.sync_copy(acc.at[pl.ds(r0, RT)], buf)
        pltpu.sync_copy(buf, out.at[c, pl.ds(r0, RT)])

    return agg_kernel


RPT = R // NT
FL = RPT * 16


def _make_scale1_kernel():
    @functools.partial(
        pl.kernel,
        mesh=_sc_mesh(),
        compiler_params=pltpu.CompilerParams(
            use_tc_tiling_on_sc=False, needs_layout_passes=False
        ),
        out_type=[
            jax.ShapeDtypeStruct((R * 16,), _f32),
            jax.ShapeDtypeStruct((R,), _f32),
        ],
        scratch_types=[
            pltpu.VMEM((FL,), _f32),
            pltpu.VMEM((RPT,), _f32),
            pltpu.VMEM((RPT,), _f32),
            pltpu.VMEM((RPT,), _f32),
        ],
    )
    def scale1(degf, xwf, t1f, disf, bxw, bd0, bd1, bdis):
        c = lax.axis_index("c")
        s = lax.axis_index("s")
        wid = c * NS + s
        e0 = wid * FL
        r0 = wid * RPT
        pltpu.sync_copy(xwf.at[pl.ds(e0, FL)], bxw)
        pltpu.sync_copy(degf.at[pl.ds(r0, RPT)], bd0)
        pltpu.sync_copy(degf.at[pl.ds(R + r0, RPT)], bd1)

        def chunk(k, carry):
            sl = pl.ds(k * 16, 16)
            d = bd0[sl] + bd1[sl] + 1.0
            i = jnp.int32(0x5F3759DF) - lax.shift_right_logical(
                plsc.bitcast(d, jnp.int32), 1
            )
            y = plsc.bitcast(i, _f32)
            half = 0.5 * d
            for _ in range(3):
                y = y * (1.5 - half * y * y)
            bdis[sl] = y
            return carry

        lax.fori_loop(0, RPT // 16, chunk, 0)

        def rowgrp(j, carry):
            dv = bdis[pl.ds(j * 16, 16)]
            base = j * 256
            for k in range(16):
                sl = pl.ds(base + k * 16, 16)
                bxw[sl] = bxw[sl] * dv[k]
            return carry

        lax.fori_loop(0, RPT // 16, rowgrp, 0)
        pltpu.sync_copy(bxw, t1f.at[pl.ds(e0, FL)])
        pltpu.sync_copy(bdis, disf.at[pl.ds(r0, RPT)])

    return scale1


def _make_combine_kernel(relu_bias):
    scratch = [
        pltpu.VMEM((FL,), _f32),
        pltpu.VMEM((FL,), _f32),
        pltpu.VMEM((FL,), _f32),
        pltpu.VMEM((RPT,), _f32),
    ]
    if relu_bias:
        scratch.append(pltpu.VMEM((16,), _f32))

    @functools.partial(
        pl.kernel,
        mesh=_sc_mesh(),
        compiler_params=pltpu.CompilerParams(
            use_tc_tiling_on_sc=False, needs_layout_passes=False
        ),
        out_type=jax.ShapeDtypeStruct((R * 16,), _f32),
        scratch_types=scratch,
    )
    def combine(pf, tf, disf, *rest):
        if relu_bias:
            bias, of, b0, b1b, bt, bd, bb = rest
        else:
            of, b0, b1b, bt, bd = rest
        c = lax.axis_index("c")
        s = lax.axis_index("s")
        wid = c * NS + s
        e0 = wid * FL
        r0 = wid * RPT
        pltpu.sync_copy(pf.at[pl.ds(e0, FL)], b0)
        pltpu.sync_copy(pf.at[pl.ds(R * 16 + e0, FL)], b1b)
        pltpu.sync_copy(tf.at[pl.ds(e0, FL)], bt)
        pltpu.sync_copy(disf.at[pl.ds(r0, RPT)], bd)
        if relu_bias:
            pltpu.sync_copy(bias, bb)
            bv = bb[...]

        def rowgrp(j, carry):
            dv = bd[pl.ds(j * 16, 16)]
            base = j * 256
            for k in range(16):
                sl = pl.ds(base + k * 16, 16)
                d = dv[k]
                v = (b0[sl] + b1b[sl] + bt[sl]) * d
                if relu_bias:
                    v = jnp.maximum(v + bv, 0.0) * d
                bt[sl] = v
            return carry

        lax.fori_loop(0, RPT // 16, rowgrp, 0)
        pltpu.sync_copy(bt, of.at[pl.ds(e0, FL)])

    return combine


_BN = 2000


def _mm1_tc(x, W1):
    def body(x_ref, w_ref, o_ref):
        o_ref[...] = jnp.dot(
            x_ref[...], w_ref[...], preferred_element_type=_f32
        )

    return pl.pallas_call(
        body,
        grid=(N // _BN,),
        in_specs=[
            pl.BlockSpec((_BN, 128), lambda i: (i, 0)),
            pl.BlockSpec((128, 16), lambda i: (0, 0)),
        ],
        out_specs=pl.BlockSpec((_BN, 16), lambda i: (i, 0)),
        out_shape=jax.ShapeDtypeStruct((N, W1.shape[1]), _f32),
    )(x, W1)


def _mm3_tc(g, W2, b2r):
    def body(g_ref, w_ref, b_ref, o_ref):
        o_ref[...] = (
            jnp.dot(g_ref[...], w_ref[...], preferred_element_type=_f32)
            + b_ref[...]
        )

    return pl.pallas_call(
        body,
        grid=(N // _BN,),
        in_specs=[
            pl.BlockSpec((_BN, 16), lambda i: (i, 0)),
            pl.BlockSpec((16, 40), lambda i: (0, 0)),
            pl.BlockSpec((1, 40), lambda i: (0, 0)),
        ],
        out_specs=pl.BlockSpec((_BN, 40), lambda i: (i, 0)),
        out_shape=jax.ShapeDtypeStruct((N, W2.shape[1]), _f32),
    )(g, W2, b2r)


_deg_kernel = _make_deg_kernel()
_agg16 = _make_agg_kernel(16)
_scale1 = _make_scale1_kernel()
_combine1 = _make_combine_kernel(relu_bias=True)
_combine2 = _make_combine_kernel(relu_bias=False)


def kernel(x, edge_index, W1, b1, W2, b2):
    eb = edge_index.astype(jnp.int32).reshape(2, NBT, B)
    zr = jnp.zeros((RT,), _f32)
    z16 = jnp.zeros((RT, 16), _f32)

    xw1 = _mm1_tc(x, W1)
    xwf = jnp.pad(xw1, ((0, R - N), (0, 0))).reshape(-1)
    deg = _deg_kernel(eb, zr)

    t1f, disf = _scale1(deg.reshape(-1), xwf)
    p = _agg16(t1f.reshape(R, 16), eb, z16)
    t2f = _combine1(p.reshape(-1), t1f, disf, b1)
    q = _agg16(t2f.reshape(R, 16), eb, z16)
    gf = _combine2(q.reshape(-1), t2f, disf)
    out = _mm3_tc(gf.reshape(R, 16), W2, b2.reshape(1, -1))
    return out
